# baseline (speedup 1.0000x reference)
import jax
import jax.numpy as jnp
from jax import lax
from jax.experimental import pallas as pl
from jax.experimental.pallas import tpu as pltpu
from jax.experimental.pallas import tpu_sc as plsc

N_NODES = 10000
N_EDGES = 320000
D_FEAT = 128
K_HEADS = 3
D_EDGE = 16
D_OUT = K_HEADS + D_EDGE
TBL_W = 2 * K_HEADS

NC = 2
NS = 16
NW = NC * NS
E_PER_W = N_EDGES // NW
CHUNK = 2000
SUB = 400
N_CHUNKS = E_PER_W // CHUNK
N_SUB = CHUNK // SUB
LANES = 16


def _project_body(x_ref, w_ref, out_ref):
    out_ref[...] = lax.dot_general(
        x_ref[...], w_ref[...], (((1,), (0,)), ((), ())),
        preferred_element_type=jnp.float32,
        precision=lax.Precision.HIGHEST)


def _project(x_0, w):
    return pl.pallas_call(
        _project_body,
        out_shape=jax.ShapeDtypeStruct((N_NODES, TBL_W), jnp.float32),
    )(x_0, w)


def _lift_body(table_hbm, src_hbm, tgt_hbm, x1_hbm, out_hbm,
               table_v, src_v, tgt_v, x1_v, out_v):
    wid = lax.axis_index("s") * NC + lax.axis_index("c")
    base = wid * E_PER_W
    pltpu.sync_copy(table_hbm, table_v)
    lane = lax.iota(jnp.int32, LANES)

    for c in range(N_CHUNKS):
        row0 = base + c * CHUNK
        pltpu.sync_copy(src_hbm.at[pl.ds(row0, CHUNK)], src_v)
        pltpu.sync_copy(tgt_hbm.at[pl.ds(row0, CHUNK)], tgt_v)

        for sb in range(N_SUB):
            e0 = sb * SUB
            pltpu.sync_copy(
                x1_hbm.at[pl.ds((row0 + e0) * D_EDGE, SUB * D_EDGE)], x1_v)

            def body(i, carry):
                g = e0 + i * LANES
                s_idx = src_v[pl.ds(g, LANES)] * TBL_W
                t_idx = tgt_v[pl.ds(g, LANES)] * TBL_W
                e19 = g * D_OUT + lane * D_OUT
                for k in range(K_HEADS):
                    a = plsc.load_gather(table_v, [s_idx + k])
                    b = plsc.load_gather(table_v, [t_idx + (K_HEADS + k)])
                    h = jnp.maximum(a + b, 0.0)
                    plsc.store_scatter(out_v, [e19 + k], h)
                for u in range(LANES):
                    row = x1_v[pl.ds((i * LANES + u) * D_EDGE, D_EDGE)]
                    plsc.store_scatter(
                        out_v, [(g + u) * D_OUT + K_HEADS + lane], row)
                return carry

            lax.fori_loop(0, SUB // LANES, body, 0)

        pltpu.sync_copy(out_v, out_hbm.at[pl.ds(row0 * D_OUT, CHUNK * D_OUT)])


def _lift(table, src, tgt, x_1):
    return pl.kernel(
        _lift_body,
        out_type=jax.ShapeDtypeStruct((N_EDGES * D_OUT,), jnp.float32),
        mesh=plsc.VectorSubcoreMesh(core_axis_name="c", subcore_axis_name="s"),
        compiler_params=pltpu.CompilerParams(needs_layout_passes=False),
        scratch_types=[
            pltpu.VMEM((N_NODES * TBL_W,), jnp.float32),
            pltpu.VMEM((CHUNK,), jnp.int32),
            pltpu.VMEM((CHUNK,), jnp.int32),
            pltpu.VMEM((SUB * D_EDGE,), jnp.float32),
            pltpu.VMEM((CHUNK * D_OUT,), jnp.float32),
        ],
    )(table, src, tgt, x_1.reshape(-1))


def kernel(x_0, x_1, neighborhood_0_to_0, att):
    idx = neighborhood_0_to_0.astype(jnp.int32)
    src, tgt = idx[0], idx[1]
    a = att[:, :, 0]
    w = jnp.concatenate([a[:, :D_FEAT].T, a[:, D_FEAT:].T], axis=1)
    table = _project(x_0, w)
    return _lift(table.reshape(-1), src, tgt, x_1).reshape(N_EDGES, D_OUT)


# speedup vs baseline: 1.1105x; 1.1105x over previous
import jax
import jax.numpy as jnp
from jax import lax
from jax.experimental import pallas as pl
from jax.experimental.pallas import tpu as pltpu
from jax.experimental.pallas import tpu_sc as plsc

N_NODES = 10000
N_EDGES = 320000
D_FEAT = 128
K_HEADS = 3
D_EDGE = 16
D_OUT = K_HEADS + D_EDGE
TBL_W = 2 * K_HEADS

NC = 2
NS = 16
NW = NC * NS
E_PER_W = N_EDGES // NW
CHUNK = 2000
SUB = 400
N_CHUNKS = E_PER_W // CHUNK
N_SUB = CHUNK // SUB
LANES = 16


def _project_body(x_ref, w_ref, out_ref):
    out_ref[...] = lax.dot_general(
        x_ref[...], w_ref[...], (((1,), (0,)), ((), ())),
        preferred_element_type=jnp.float32,
        precision=lax.Precision.HIGHEST)


def _project(x_0, w):
    return pl.pallas_call(
        _project_body,
        out_shape=jax.ShapeDtypeStruct((N_NODES, TBL_W), jnp.float32),
    )(x_0, w)


def _lift_body(table_hbm, src_hbm, tgt_hbm, x1_hbm, out_hbm,
               table_v, src_v, tgt_v, x1_v, out_v):
    wid = lax.axis_index("s") * NC + lax.axis_index("c")
    base = wid * E_PER_W
    pltpu.sync_copy(table_hbm, table_v)
    lane = lax.iota(jnp.int32, LANES)

    for c in range(N_CHUNKS):
        row0 = base + c * CHUNK
        pltpu.sync_copy(src_hbm.at[pl.ds(row0, CHUNK)], src_v)
        pltpu.sync_copy(tgt_hbm.at[pl.ds(row0, CHUNK)], tgt_v)

        for sb in range(N_SUB):
            e0 = sb * SUB
            pltpu.sync_copy(x1_hbm.at[pl.ds(row0 + e0, SUB)], x1_v)

            def body(i, carry):
                g = e0 + i * LANES
                s_idx = src_v[pl.ds(g, LANES)] * TBL_W
                t_idx = tgt_v[pl.ds(g, LANES)] * TBL_W
                rows = g + lane
                k_col = jnp.zeros((LANES,), jnp.int32)
                for k in range(K_HEADS):
                    a = plsc.load_gather(table_v, [s_idx + k])
                    b = plsc.load_gather(table_v, [t_idx + (K_HEADS + k)])
                    h = jnp.maximum(a + b, 0.0)
                    plsc.store_scatter(out_v, [rows, k_col + k], h)
                for u in range(LANES):
                    row = x1_v[i * LANES + u]
                    plsc.store_scatter(
                        out_v,
                        [jnp.full((LANES,), g + u, jnp.int32), K_HEADS + lane],
                        row)
                return carry

            lax.fori_loop(0, SUB // LANES, body, 0)

        pltpu.sync_copy(out_v, out_hbm.at[pl.ds(row0, CHUNK)])


def _lift(table, src, tgt, x_1):
    return pl.kernel(
        _lift_body,
        out_type=jax.ShapeDtypeStruct((N_EDGES, D_OUT), jnp.float32),
        mesh=plsc.VectorSubcoreMesh(core_axis_name="c", subcore_axis_name="s"),
        compiler_params=pltpu.CompilerParams(
            needs_layout_passes=False, use_tc_tiling_on_sc=False),
        scratch_types=[
            pltpu.VMEM((N_NODES * TBL_W,), jnp.float32),
            pltpu.VMEM((CHUNK,), jnp.int32),
            pltpu.VMEM((CHUNK,), jnp.int32),
            pltpu.VMEM((SUB, D_EDGE), jnp.float32),
            pltpu.VMEM((CHUNK, D_OUT), jnp.float32),
        ],
    )(table, src, tgt, x_1)


def kernel(x_0, x_1, neighborhood_0_to_0, att):
    idx = neighborhood_0_to_0.astype(jnp.int32)
    src, tgt = idx[0], idx[1]
    a = att[:, :, 0]
    w = jnp.concatenate([a[:, :D_FEAT].T, a[:, D_FEAT:].T], axis=1)
    table = _project(x_0, w)
    return _lift(table.reshape(-1), src, tgt, x_1)


# speedup vs baseline: 4.0747x; 3.6692x over previous
import jax
import jax.numpy as jnp
from jax import lax
from jax.experimental import pallas as pl
from jax.experimental.pallas import tpu as pltpu
from jax.experimental.pallas import tpu_sc as plsc

N_NODES = 10000
N_EDGES = 320000
D_FEAT = 128
K_HEADS = 3
D_EDGE = 16
D_OUT = K_HEADS + D_EDGE
TBL_W = 2 * K_HEADS

NC = 2
NS = 16
NW = NC * NS
E_PER_W = N_EDGES // NW
CHUNK = 400
N_CHUNKS = E_PER_W // CHUNK
LANES = 16
NBUF = 2


def _project_body(x_ref, w_ref, out_ref):
    out_ref[...] = lax.dot_general(
        x_ref[...], w_ref[...], (((1,), (0,)), ((), ())),
        preferred_element_type=jnp.float32,
        precision=lax.Precision.HIGHEST)


def _project(x_0, w):
    return pl.pallas_call(
        _project_body,
        out_shape=jax.ShapeDtypeStruct((N_NODES, TBL_W), jnp.float32),
    )(x_0, w)


def _lift_body(table_hbm, src_hbm, tgt_hbm, x1t_hbm, out_hbm,
               table_v, src_v, tgt_v, out_v, in_sem, x1_sem, out_sem):
    wid = lax.axis_index("s") * NC + lax.axis_index("c")
    base = wid * E_PER_W

    def start_in(c, slot):
        row0 = base + c * CHUNK
        pltpu.async_copy(src_hbm.at[pl.ds(row0, CHUNK)],
                         src_v.at[slot], in_sem[slot])
        pltpu.async_copy(tgt_hbm.at[pl.ds(row0, CHUNK)],
                         tgt_v.at[slot], in_sem[slot])
        pltpu.async_copy(x1t_hbm.at[:, pl.ds(row0, CHUNK)],
                         out_v.at[slot].at[pl.ds(K_HEADS, D_EDGE), :],
                         x1_sem[slot])

    tbl_cp = pltpu.async_copy(table_hbm, table_v, out_sem[0])
    start_in(0, 0)
    tbl_cp.wait()

    for c in range(N_CHUNKS):
        slot = c % NBUF
        row0 = base + c * CHUNK
        pltpu.make_async_copy(src_hbm.at[pl.ds(0, CHUNK)],
                              src_v.at[slot], in_sem[slot]).wait()
        pltpu.make_async_copy(tgt_hbm.at[pl.ds(0, CHUNK)],
                              tgt_v.at[slot], in_sem[slot]).wait()

        def body(i, carry):
            g = i * LANES
            s_idx = src_v[slot, pl.ds(g, LANES)] * TBL_W
            t_idx = tgt_v[slot, pl.ds(g, LANES)] * TBL_W
            for k in range(K_HEADS):
                a = plsc.load_gather(table_v, [s_idx + k])
                b = plsc.load_gather(table_v, [t_idx + (K_HEADS + k)])
                out_v[slot, k, pl.ds(g, LANES)] = jnp.maximum(a + b, 0.0)
            return carry

        lax.fori_loop(0, CHUNK // LANES, body, 0)

        pltpu.make_async_copy(x1t_hbm.at[:, pl.ds(0, CHUNK)],
                              out_v.at[slot].at[pl.ds(K_HEADS, D_EDGE), :],
                              x1_sem[slot]).wait()
        pltpu.async_copy(out_v.at[slot], out_hbm.at[:, pl.ds(row0, CHUNK)],
                         out_sem[slot])
        if c + 1 < N_CHUNKS:
            nxt = c + 1
            if nxt >= NBUF:
                pltpu.make_async_copy(
                    out_v.at[nxt % NBUF],
                    out_hbm.at[:, pl.ds(base + (nxt - NBUF) * CHUNK, CHUNK)],
                    out_sem[nxt % NBUF]).wait()
            start_in(nxt, nxt % NBUF)

    for c in range(N_CHUNKS - NBUF, N_CHUNKS):
        slot = c % NBUF
        pltpu.make_async_copy(out_v.at[slot],
                              out_hbm.at[:, pl.ds(base + c * CHUNK, CHUNK)],
                              out_sem[slot]).wait()


def _lift(table, src, tgt, x_1t):
    return pl.kernel(
        _lift_body,
        out_type=jax.ShapeDtypeStruct((D_OUT, N_EDGES), jnp.float32),
        mesh=plsc.VectorSubcoreMesh(core_axis_name="c", subcore_axis_name="s"),
        compiler_params=pltpu.CompilerParams(
            needs_layout_passes=False, use_tc_tiling_on_sc=False),
        scratch_types=[
            pltpu.VMEM((N_NODES * TBL_W,), jnp.float32),
            pltpu.VMEM((NBUF, CHUNK), jnp.int32),
            pltpu.VMEM((NBUF, CHUNK), jnp.int32),
            pltpu.VMEM((NBUF, D_OUT, CHUNK), jnp.float32),
            [pltpu.SemaphoreType.DMA] * NBUF,
            [pltpu.SemaphoreType.DMA] * NBUF,
            [pltpu.SemaphoreType.DMA] * NBUF,
        ],
    )(table, src, tgt, x_1t)


def kernel(x_0, x_1, neighborhood_0_to_0, att):
    idx = neighborhood_0_to_0.astype(jnp.int32)
    src, tgt = idx[0], idx[1]
    a = att[:, :, 0]
    w = jnp.concatenate([a[:, :D_FEAT].T, a[:, D_FEAT:].T], axis=1)
    table = _project(x_0, w)
    out_t = _lift(table.reshape(-1), src, tgt, x_1.T)
    return out_t.T


# speedup vs baseline: 4.3647x; 1.0712x over previous
import jax
import jax.numpy as jnp
from jax import lax
from jax.experimental import pallas as pl
from jax.experimental.pallas import tpu as pltpu
from jax.experimental.pallas import tpu_sc as plsc

N_NODES = 10000
N_EDGES = 320000
D_FEAT = 128
K_HEADS = 3
D_EDGE = 16
D_OUT = K_HEADS + D_EDGE
TBL_W = 2 * K_HEADS

NC = 2
NS = 16
NW = NC * NS
E_PER_W = N_EDGES // NW
CHUNK = 400
N_CHUNKS = E_PER_W // CHUNK
LANES = 16
NBUF = 2


def _project_body(x_ref, w_ref, out_ref):
    out_ref[...] = lax.dot_general(
        w_ref[...], x_ref[...], (((0,), (1,)), ((), ())),
        preferred_element_type=jnp.float32,
        precision=lax.Precision.HIGHEST)


def _project(x_0, w):
    return pl.pallas_call(
        _project_body,
        out_shape=jax.ShapeDtypeStruct((TBL_W, N_NODES), jnp.float32),
    )(x_0, w)


def _lift_body(table_hbm, src_hbm, tgt_hbm, x1t_hbm, out_hbm,
               table_v, src_v, tgt_v, out_v, in_sem, x1_sem, out_sem):
    wid = lax.axis_index("s") * NC + lax.axis_index("c")
    base = wid * E_PER_W

    def start_in(c, slot):
        row0 = base + c * CHUNK
        pltpu.async_copy(src_hbm.at[pl.ds(row0, CHUNK)],
                         src_v.at[slot], in_sem[slot])
        pltpu.async_copy(tgt_hbm.at[pl.ds(row0, CHUNK)],
                         tgt_v.at[slot], in_sem[slot])
        pltpu.async_copy(x1t_hbm.at[:, pl.ds(row0, CHUNK)],
                         out_v.at[slot].at[pl.ds(K_HEADS, D_EDGE), :],
                         x1_sem[slot])

    tbl_cp = pltpu.async_copy(table_hbm, table_v, out_sem[0])
    start_in(0, 0)
    tbl_cp.wait()

    for c in range(N_CHUNKS):
        slot = c % NBUF
        row0 = base + c * CHUNK
        pltpu.make_async_copy(src_hbm.at[pl.ds(0, CHUNK)],
                              src_v.at[slot], in_sem[slot]).wait()
        pltpu.make_async_copy(tgt_hbm.at[pl.ds(0, CHUNK)],
                              tgt_v.at[slot], in_sem[slot]).wait()

        def body(i, carry):
            g = i * LANES
            s_idx = src_v[slot, pl.ds(g, LANES)]
            t_idx = tgt_v[slot, pl.ds(g, LANES)]
            for k in range(K_HEADS):
                a = plsc.load_gather(table_v, [s_idx + k * N_NODES])
                b = plsc.load_gather(table_v,
                                     [t_idx + (K_HEADS + k) * N_NODES])
                out_v[slot, k, pl.ds(g, LANES)] = jnp.maximum(a + b, 0.0)
            return carry

        lax.fori_loop(0, CHUNK // LANES, body, 0)

        pltpu.make_async_copy(x1t_hbm.at[:, pl.ds(0, CHUNK)],
                              out_v.at[slot].at[pl.ds(K_HEADS, D_EDGE), :],
                              x1_sem[slot]).wait()
        pltpu.async_copy(out_v.at[slot], out_hbm.at[:, pl.ds(row0, CHUNK)],
                         out_sem[slot])
        if c + 1 < N_CHUNKS:
            nxt = c + 1
            if nxt >= NBUF:
                pltpu.make_async_copy(
                    out_v.at[nxt % NBUF],
                    out_hbm.at[:, pl.ds(base + (nxt - NBUF) * CHUNK, CHUNK)],
                    out_sem[nxt % NBUF]).wait()
            start_in(nxt, nxt % NBUF)

    for c in range(N_CHUNKS - NBUF, N_CHUNKS):
        slot = c % NBUF
        pltpu.make_async_copy(out_v.at[slot],
                              out_hbm.at[:, pl.ds(base + c * CHUNK, CHUNK)],
                              out_sem[slot]).wait()


def _lift(table, src, tgt, x_1t):
    return pl.kernel(
        _lift_body,
        out_type=jax.ShapeDtypeStruct((D_OUT, N_EDGES), jnp.float32),
        mesh=plsc.VectorSubcoreMesh(core_axis_name="c", subcore_axis_name="s"),
        compiler_params=pltpu.CompilerParams(
            needs_layout_passes=False, use_tc_tiling_on_sc=False),
        scratch_types=[
            pltpu.VMEM((N_NODES * TBL_W,), jnp.float32),
            pltpu.VMEM((NBUF, CHUNK), jnp.int32),
            pltpu.VMEM((NBUF, CHUNK), jnp.int32),
            pltpu.VMEM((NBUF, D_OUT, CHUNK), jnp.float32),
            [pltpu.SemaphoreType.DMA] * NBUF,
            [pltpu.SemaphoreType.DMA] * NBUF,
            [pltpu.SemaphoreType.DMA] * NBUF,
        ],
    )(table, src, tgt, x_1t)


def kernel(x_0, x_1, neighborhood_0_to_0, att):
    idx = neighborhood_0_to_0.astype(jnp.int32)
    src, tgt = idx[0], idx[1]
    a = att[:, :, 0]
    w = jnp.concatenate([a[:, :D_FEAT].T, a[:, D_FEAT:].T], axis=1)
    table = _project(x_0, w)
    out_t = _lift(table.reshape(-1), src, tgt, x_1.T)
    return out_t.T
